# baseline (speedup 1.0000x reference)
import functools

import jax
import jax.numpy as jnp
from jax import lax
from jax.experimental import pallas as pl
from jax.experimental.pallas import tpu as pltpu
from jax.experimental.pallas import tpu_sc as plsc

NUM_FIELDS = 26
ROWS_PER_TABLE = 100001
D_MODEL = 128
BATCH = 16384

_info = plsc.get_sparse_core_info()
NC = _info.num_cores
NS = _info.num_subcores
LANES = _info.num_lanes
NW = NC * NS
BPW = BATCH // NW
NB = 128
NBLK = BPW // NB

_mesh = plsc.VectorSubcoreMesh(core_axis_name="c", subcore_axis_name="s")


@functools.partial(
    pl.kernel,
    mesh=_mesh,
    out_type=jax.ShapeDtypeStruct((BATCH, D_MODEL), jnp.float32),
    scratch_types=[
        pltpu.VMEM((NUM_FIELDS, BPW), jnp.int32),
        pltpu.VMEM((2, NB, D_MODEL), jnp.float32),
        pltpu.VMEM((NB, D_MODEL), jnp.float32),
        pltpu.SemaphoreType.DMA,
        pltpu.SemaphoreType.DMA,
        pltpu.SemaphoreType.DMA,
    ],
)
def _emb_kernel(idx_hbm, tab_hbm, out_hbm, idx_v, gbuf, acc, semA, semB, semC):
    wid = lax.axis_index("s") * NC + lax.axis_index("c")
    base = wid * BPW
    pltpu.sync_copy(idx_hbm.at[:, pl.ds(base, BPW)], idx_v)

    def accum(f, blk):
        buf = gbuf.at[f % 2]

        def body(r, carry):
            for c in range(D_MODEL // LANES):
                v = buf[r, pl.ds(c * LANES, LANES)]
                plsc.addupdate(acc.at[r, pl.ds(c * LANES, LANES)], v)
            return carry

        lax.fori_loop(0, NB, body, 0)

    def block_body(blk, carry):
        off = blk * NB
        cp_acc = pltpu.async_copy(
            tab_hbm.at[idx_v.at[0, pl.ds(off, NB)]], acc, semC)
        sems = (semA, semB)
        cps = [None] * NUM_FIELDS
        cps[1] = pltpu.async_copy(
            tab_hbm.at[idx_v.at[1, pl.ds(off, NB)]], gbuf.at[1 % 2], sems[1 % 2])
        cp_acc.wait()
        for f in range(1, NUM_FIELDS):
            if f + 1 < NUM_FIELDS:
                cps[f + 1] = pltpu.async_copy(
                    tab_hbm.at[idx_v.at[f + 1, pl.ds(off, NB)]],
                    gbuf.at[(f + 1) % 2], sems[(f + 1) % 2])
            cps[f].wait()
            accum(f, blk)
        pltpu.sync_copy(acc, out_hbm.at[pl.ds(base + off, NB)])
        return carry

    lax.fori_loop(0, NBLK, block_body, 0)


def kernel(x, tables):
    flat_tables = tables.reshape(NUM_FIELDS * ROWS_PER_TABLE, D_MODEL)
    offs = jnp.arange(NUM_FIELDS, dtype=jnp.int32) * ROWS_PER_TABLE
    flat_idx = (x + offs[None, :]).T
    return _emb_kernel(flat_idx, flat_tables)


# speedup vs baseline: 6.9802x; 6.9802x over previous
import functools

import jax
import jax.numpy as jnp
from jax import lax
from jax.experimental import pallas as pl
from jax.experimental.pallas import tpu as pltpu
from jax.experimental.pallas import tpu_sc as plsc

NUM_FIELDS = 26
ROWS_PER_TABLE = 100001
D_MODEL = 128
BATCH = 16384

_info = plsc.get_sparse_core_info()
NC = _info.num_cores
NS = _info.num_subcores
LANES = _info.num_lanes
NW = NC * NS
BPW = BATCH // NW
NB = 128
NBLK = BPW // NB

_mesh = plsc.VectorSubcoreMesh(core_axis_name="c", subcore_axis_name="s")


@functools.partial(
    pl.kernel,
    mesh=_mesh,
    out_type=jax.ShapeDtypeStruct((BATCH, D_MODEL), jnp.float32),
    scratch_types=[
        pltpu.VMEM((NUM_FIELDS, BPW), jnp.int32),
        pltpu.VMEM((2, NB, D_MODEL), jnp.float32),
        pltpu.VMEM((NB, D_MODEL), jnp.float32),
        pltpu.SemaphoreType.DMA,
        pltpu.SemaphoreType.DMA,
        pltpu.SemaphoreType.DMA,
    ],
)
def _emb_kernel(idx_hbm, tab_hbm, out_hbm, idx_v, gbuf, acc, semA, semB, semC):
    wid = lax.axis_index("s") * NC + lax.axis_index("c")
    base = wid * BPW
    pltpu.sync_copy(idx_hbm.at[:, pl.ds(base, BPW)], idx_v)

    def accum(f, blk):
        buf = gbuf.at[f % 2]

        def body(r, carry):
            for c in range(D_MODEL // LANES):
                v = buf[r, pl.ds(c * LANES, LANES)]
                plsc.addupdate(acc.at[r, pl.ds(c * LANES, LANES)], v)
            return carry

        lax.fori_loop(0, NB, body, 0)

    def block_body(blk, carry):
        off = blk * NB
        cp_acc = pltpu.async_copy(
            tab_hbm.at[0].at[idx_v.at[0, pl.ds(off, NB)]], acc, semC)
        sems = (semA, semB)
        cps = [None] * NUM_FIELDS
        cps[1] = pltpu.async_copy(
            tab_hbm.at[1].at[idx_v.at[1, pl.ds(off, NB)]], gbuf.at[1 % 2],
            sems[1 % 2])
        cp_acc.wait()
        for f in range(1, NUM_FIELDS):
            if f + 1 < NUM_FIELDS:
                cps[f + 1] = pltpu.async_copy(
                    tab_hbm.at[f + 1].at[idx_v.at[f + 1, pl.ds(off, NB)]],
                    gbuf.at[(f + 1) % 2], sems[(f + 1) % 2])
            cps[f].wait()
            accum(f, blk)
        pltpu.sync_copy(acc, out_hbm.at[pl.ds(base + off, NB)])
        return carry

    lax.fori_loop(0, NBLK, block_body, 0)


def kernel(x, tables):
    idx_t = x.T
    return _emb_kernel(idx_t, tables)
